# baseline (speedup 1.0000x reference)
import functools
import math

import jax
import jax.numpy as jnp
from jax import lax
from jax.experimental import pallas as pl
from jax.experimental.pallas import tpu as pltpu

_ROW_TILE = 256


def _atten_pool_kernel(x_full_ref, x_cur_ref, x_prev_ref,
                       segc_prev_ref, segr_ref,
                       wq_ref, bq_ref, wk_ref, bk_ref, wv_ref, bv_ref,
                       ws_ref, bs_ref,
                       out_ref, k_ref, v_ref, s_buf_ref,
                       *, num_segments, num_tiles, scale):
    i = pl.program_id(0)

    @pl.when(i == 0)
    def _init():
        x_full = x_full_ref[:]
        k = jnp.dot(x_full, wk_ref[:],
                    preferred_element_type=jnp.float32) + bk_ref[:]
        v = jnp.dot(x_full, wv_ref[:],
                    preferred_element_type=jnp.float32) + bv_ref[:]
        k_ref[:] = k.astype(jnp.bfloat16)
        v_ref[:] = v.astype(jnp.bfloat16)
        out_ref[:] = jnp.full_like(out_ref, -jnp.inf)

    @pl.when(i < num_tiles)
    def _scores():
        x_t = x_cur_ref[:]
        q = (jnp.dot(x_t, wq_ref[:],
                     preferred_element_type=jnp.float32) + bq_ref[:]) * scale
        s = lax.dot_general(q.astype(jnp.bfloat16), k_ref[:],
                            (((1,), (1,)), ((), ())),
                            preferred_element_type=jnp.float32)
        s_buf_ref[pl.ds(jax.lax.rem(i, 2), 1)] = s[None]

    @pl.when(i > 0)
    def _softmax_pool():
        s = s_buf_ref[pl.ds(jax.lax.rem(i - 1, 2), 1)][0]
        seg_c = segc_prev_ref[0]
        seg_r = segr_ref[:]
        mask = seg_c == seg_r
        s = jnp.where(mask, s, -jnp.inf)
        m = jnp.max(s, axis=1, keepdims=True)
        p = jnp.exp2(s - m)
        denom = jnp.sum(p, axis=1, keepdims=True)

        o = jnp.dot(p.astype(jnp.bfloat16), v_ref[:],
                    preferred_element_type=jnp.float32) * (1.0 / denom)
        o = o + jnp.dot(x_prev_ref[:], ws_ref[:],
                        preferred_element_type=jnp.float32) + bs_ref[:]

        first = jnp.min(seg_c)
        last = jnp.max(seg_c)
        for b in range(num_segments):
            @pl.when((b >= first) & (b <= last))
            def _pool():
                mb = seg_c == b
                pb = jnp.max(jnp.where(mb, o, -jnp.inf), axis=0,
                             keepdims=True)
                out_ref[b:b + 1, :] = jnp.maximum(out_ref[b:b + 1, :], pb)


def kernel(x, subgbatch, Wq, bq, Wk, bk, Wv, bv, Wskip, bskip):
    n, d = x.shape
    c = Wq.shape[1]
    num_segments = 16
    t = _ROW_TILE
    num_tiles = n // t
    seg = subgbatch.astype(jnp.int32)
    segc = seg.reshape(num_tiles, t, 1)
    segr = seg.reshape(1, n)

    cur = lambda i: (jnp.minimum(i, num_tiles - 1), 0)
    prev = lambda i: (jnp.maximum(i - 1, 0), 0)

    fn = pl.pallas_call(
        functools.partial(_atten_pool_kernel, num_segments=num_segments,
                          num_tiles=num_tiles,
                          scale=math.log2(math.e) / math.sqrt(c)),
        grid=(num_tiles + 1,),
        in_specs=[
            pl.BlockSpec((n, d), lambda i: (0, 0)),
            pl.BlockSpec((t, d), cur),
            pl.BlockSpec((t, d), prev),
            pl.BlockSpec((1, t, 1), lambda i: (jnp.maximum(i - 1, 0), 0, 0)),
            pl.BlockSpec((1, n), lambda i: (0, 0)),
            pl.BlockSpec((d, c), lambda i: (0, 0)),
            pl.BlockSpec((1, c), lambda i: (0, 0)),
            pl.BlockSpec((d, c), lambda i: (0, 0)),
            pl.BlockSpec((1, c), lambda i: (0, 0)),
            pl.BlockSpec((d, c), lambda i: (0, 0)),
            pl.BlockSpec((1, c), lambda i: (0, 0)),
            pl.BlockSpec((d, c), lambda i: (0, 0)),
            pl.BlockSpec((1, c), lambda i: (0, 0)),
        ],
        out_specs=pl.BlockSpec((num_segments, c), lambda i: (0, 0)),
        scratch_shapes=[
            pltpu.VMEM((n, c), jnp.bfloat16),
            pltpu.VMEM((n, c), jnp.bfloat16),
            pltpu.VMEM((2, t, n), jnp.float32),
        ],
        out_shape=jax.ShapeDtypeStruct((num_segments, c), jnp.float32),
    )
    return fn(x, x, x, segc, segr,
              Wq, bq.reshape(1, c), Wk, bk.reshape(1, c),
              Wv, bv.reshape(1, c), Wskip, bskip.reshape(1, c))


# speedup vs baseline: 1.0776x; 1.0776x over previous
import functools
import math

import jax
import jax.numpy as jnp
from jax import lax
from jax.experimental import pallas as pl
from jax.experimental.pallas import tpu as pltpu

_ROW_TILE = 256


def _atten_pool_kernel(x_full_ref, x_tile_ref, segc_ref, segr_ref,
                       wq_ref, bq_ref, wk_ref, bk_ref, wv_ref, bv_ref,
                       ws_ref, bs_ref,
                       out_ref, k_ref, v_ref, *, num_segments, scale):
    i = pl.program_id(0)

    @pl.when(i == 0)
    def _init():
        x_full = x_full_ref[:]
        k = jnp.dot(x_full, wk_ref[:],
                    preferred_element_type=jnp.float32) + bk_ref[:]
        v = jnp.dot(x_full, wv_ref[:],
                    preferred_element_type=jnp.float32) + bv_ref[:]
        k_ref[:] = k.astype(jnp.bfloat16)
        v_ref[:] = v.astype(jnp.bfloat16)
        out_ref[:] = jnp.full_like(out_ref, -jnp.inf)

    x_t = x_tile_ref[:]
    q = (jnp.dot(x_t, wq_ref[:],
                 preferred_element_type=jnp.float32) + bq_ref[:]) * scale

    s = lax.dot_general(q.astype(jnp.bfloat16), k_ref[:],
                        (((1,), (1,)), ((), ())),
                        preferred_element_type=jnp.float32)
    seg_c = segc_ref[0]
    seg_r = segr_ref[:]
    mask = seg_c == seg_r
    s = jnp.where(mask, s, -jnp.inf)
    m = jnp.max(s, axis=1, keepdims=True)
    p = jnp.exp2(s - m)
    denom = jnp.sum(p, axis=1, keepdims=True)

    o = jnp.dot(p.astype(jnp.bfloat16), v_ref[:],
                preferred_element_type=jnp.float32) * (1.0 / denom)
    o = o + jnp.dot(x_t, ws_ref[:],
                    preferred_element_type=jnp.float32) + bs_ref[:]

    first = jnp.min(seg_c)
    last = jnp.max(seg_c)
    for b in range(num_segments):
        @pl.when((b >= first) & (b <= last))
        def _pool():
            mb = seg_c == b
            pb = jnp.max(jnp.where(mb, o, -jnp.inf), axis=0,
                         keepdims=True)
            out_ref[b:b + 1, :] = jnp.maximum(out_ref[b:b + 1, :], pb)


def kernel(x, subgbatch, Wq, bq, Wk, bk, Wv, bv, Wskip, bskip):
    n, d = x.shape
    c = Wq.shape[1]
    num_segments = 16
    t = _ROW_TILE
    num_tiles = n // t
    seg = subgbatch.astype(jnp.int32)
    segc = seg.reshape(num_tiles, t, 1)
    segr = seg.reshape(1, n)

    fn = pl.pallas_call(
        functools.partial(_atten_pool_kernel, num_segments=num_segments,
                          scale=math.log2(math.e) / math.sqrt(c)),
        grid=(num_tiles,),
        in_specs=[
            pl.BlockSpec((n, d), lambda i: (0, 0)),
            pl.BlockSpec((t, d), lambda i: (i, 0)),
            pl.BlockSpec((1, t, 1), lambda i: (i, 0, 0)),
            pl.BlockSpec((1, n), lambda i: (0, 0)),
            pl.BlockSpec((d, c), lambda i: (0, 0)),
            pl.BlockSpec((1, c), lambda i: (0, 0)),
            pl.BlockSpec((d, c), lambda i: (0, 0)),
            pl.BlockSpec((1, c), lambda i: (0, 0)),
            pl.BlockSpec((d, c), lambda i: (0, 0)),
            pl.BlockSpec((1, c), lambda i: (0, 0)),
            pl.BlockSpec((d, c), lambda i: (0, 0)),
            pl.BlockSpec((1, c), lambda i: (0, 0)),
        ],
        out_specs=pl.BlockSpec((num_segments, c), lambda i: (0, 0)),
        scratch_shapes=[
            pltpu.VMEM((n, c), jnp.bfloat16),
            pltpu.VMEM((n, c), jnp.bfloat16),
        ],
        out_shape=jax.ShapeDtypeStruct((num_segments, c), jnp.float32),
    )
    return fn(x, x, segc, segr,
              Wq, bq.reshape(1, c), Wk, bk.reshape(1, c),
              Wv, bv.reshape(1, c), Wskip, bskip.reshape(1, c))
